# Initial kernel scaffold; baseline (speedup 1.0000x reference)
import jax
import jax.numpy as jnp
from jax.experimental import pallas as pl


def kernel(corr_index, speed_index, angle_index, nei_index, hidden_state, cn, params):
    raise NotImplementedError("write your pallas kernel here")


# speedup vs baseline: 12.8590x; 12.8590x over previous
import jax
import jax.numpy as jnp
from jax.experimental import pallas as pl

N = 32
D = 64
HEADS = 4
OUT = 3 * D
HD = OUT // HEADS
M = N * N
_EPS = 1e-5
_NEG = -1e30


def _ln(x, w, b):
    u = jnp.mean(x, axis=-1, keepdims=True)
    xc = x - u
    s = jnp.mean(xc * xc, axis=-1, keepdims=True)
    return w * (xc * jax.lax.rsqrt(s + _EPS)) + b


def _fused_kernel(corr_ref, speed_ref, angle_ref, maskc_ref, maskr_ref,
                  hs_ref, cn_ref,
                  wr_ref, br_ref, lnwr_ref, lnbr_ref,
                  wsa_ref, bsa_ref, lnwsa_ref, lnbsa_ref,
                  wng_ref, bng_ref, lnwng_ref, lnbng_ref,
                  wq_ref, bq_ref, wk_ref, bk_ref, wv_ref, bv_ref,
                  wmg1_ref, bmg1_ref, wmg2_ref, bmg2_ref,
                  wfc_ref, bfc_ref,
                  ww_ref, bw_ref, lnww_ref, lnbw_ref,
                  hout_ref, cout_ref):
    corr = corr_ref[...]
    speed = speed_ref[...]
    angle = angle_ref[...]
    mask_c = maskc_ref[...]
    mask_r = maskr_ref[...]
    hs = hs_ref[...]

    m_col = jax.lax.broadcasted_iota(jnp.int32, (M, N), 0)
    j_col = jax.lax.broadcasted_iota(jnp.int32, (M, N), 1)
    tile_m = (jnp.remainder(m_col, N) == j_col).astype(jnp.float32)
    sel = ((m_col // N) == j_col).astype(jnp.float32)
    i_row = jax.lax.broadcasted_iota(jnp.int32, (N, M), 0)
    m_row = jax.lax.broadcasted_iota(jnp.int32, (N, M), 1)
    selt = (i_row == (m_row // N)).astype(jnp.float32)

    inp = jnp.dot(tile_m, hs, preferred_element_type=jnp.float32)
    hi = jnp.dot(sel, hs, preferred_element_type=jnp.float32)

    r_t = jnp.maximum(
        _ln(corr[:, 0:1] * wr_ref[0:1, :] + corr[:, 1:2] * wr_ref[1:2, :]
            + br_ref[...], lnwr_ref[...], lnbr_ref[...]), 0.0)
    s_t = jnp.maximum(
        _ln(speed * wsa_ref[...] + bsa_ref[...],
            lnwsa_ref[...], lnbsa_ref[...]), 0.0)
    a_t = jnp.maximum(
        _ln(angle * wsa_ref[...] + bsa_ref[...],
            lnwsa_ref[...], lnbsa_ref[...]), 0.0)

    parts = (r_t, s_t, a_t, hi, inp)

    acc = jnp.dot(parts[0], wng_ref[0], preferred_element_type=jnp.float32)
    for p in range(1, 5):
        acc = acc + jnp.dot(parts[p], wng_ref[p],
                            preferred_element_type=jnp.float32)
    ngate = jax.nn.sigmoid(_ln(acc + bng_ref[...],
                               lnwng_ref[...], lnbng_ref[...]))

    mg_h = jnp.maximum(
        speed * wmg1_ref[0:1, :] + angle * wmg1_ref[1:2, :] + bmg1_ref[...],
        0.0)
    mg = jax.nn.sigmoid(
        jnp.sum(mg_h * wmg2_ref[...], axis=1, keepdims=True)
        + bmg2_ref[...])

    scale = 1.0 / (HD ** 0.5)
    tt = jnp.zeros((M, 1), jnp.float32) + bfc_ref[...]
    for h in range(HEADS):
        qh = jnp.dot(parts[0], wq_ref[0, h], preferred_element_type=jnp.float32)
        kh = jnp.dot(parts[0], wk_ref[0, h], preferred_element_type=jnp.float32)
        vh = jnp.dot(parts[0], wv_ref[0, h], preferred_element_type=jnp.float32)
        for p in range(1, 5):
            qh = qh + jnp.dot(parts[p], wq_ref[p, h],
                              preferred_element_type=jnp.float32)
            kh = kh + jnp.dot(parts[p], wk_ref[p, h],
                              preferred_element_type=jnp.float32)
            vh = vh + jnp.dot(parts[p], wv_ref[p, h],
                              preferred_element_type=jnp.float32)
        qh = qh + bq_ref[h]
        kh = kh + bk_ref[h]
        vh = vh + bv_ref[h]
        s = jax.lax.dot_general(qh, kh, (((1,), (1,)), ((), ())),
                                preferred_element_type=jnp.float32) * scale
        s = jnp.where(mask_r > 0, s, _NEG)
        mx = jnp.max(s, axis=1, keepdims=True)
        e = jnp.exp(s - mx)
        den = jnp.sum(e, axis=1, keepdims=True)
        p_att = e / den
        vg = vh * mg
        oh = jnp.dot(p_att, vg, preferred_element_type=jnp.float32)
        tt = tt + jnp.sum(oh * wfc_ref[h], axis=1, keepdims=True)

    pos0 = mask_c * tt
    pos = jnp.where(pos0 == 0.0, -10000.0, pos0)
    num = jnp.exp(pos)
    den_seg = jnp.dot(selt, num, preferred_element_type=jnp.float32)
    den_flat = jnp.dot(sel, den_seg, preferred_element_type=jnp.float32)
    pos_t = num / jnp.maximum(den_flat, 1e-30)

    hv = inp * ngate * pos_t
    hfull = mask_c * hv
    hsum = jnp.dot(selt, hfull, preferred_element_type=jnp.float32)
    hsum = jnp.maximum(
        _ln(jnp.dot(hsum, ww_ref[...], preferred_element_type=jnp.float32)
            + bw_ref[...], lnww_ref[...], lnbw_ref[...]), 0.0)
    c = hsum + cn_ref[...]
    cout_ref[...] = c
    hout_ref[...] = hs + jnp.tanh(c)


def _run(corr_t, speed_t, angle_t, mask_c, mask_r, hidden_state, cn, p,
         interpret=False):
    row2 = lambda v: v.reshape(1, -1)
    wq = p['W_q'].reshape(5, D, HEADS, HD).transpose(0, 2, 1, 3)
    wk = p['W_k'].reshape(5, D, HEADS, HD).transpose(0, 2, 1, 3)
    wv = p['W_v'].reshape(5, D, HEADS, HD).transpose(0, 2, 1, 3)
    bq = p['b_q'].reshape(HEADS, 1, HD)
    bk = p['b_k'].reshape(HEADS, 1, HD)
    bv = p['b_v'].reshape(HEADS, 1, HD)
    wfc = p['W_fc'][:, 0].reshape(HEADS, 1, HD)
    out_sds = (jax.ShapeDtypeStruct((N, D), jnp.float32),
               jax.ShapeDtypeStruct((N, D), jnp.float32))
    return pl.pallas_call(_fused_kernel, out_shape=out_sds,
                          interpret=interpret)(
        corr_t, speed_t, angle_t, mask_c, mask_r, hidden_state, cn,
        p['W_r'], row2(p['b_r']), row2(p['lnw_r']), row2(p['lnb_r']),
        p['W_sa'], row2(p['b_sa']), row2(p['lnw_sa']), row2(p['lnb_sa']),
        p['W_ngate'].reshape(5, D, D), row2(p['b_ngate']),
        row2(p['lnw_ngate']), row2(p['lnb_ngate']),
        wq, bq, wk, bk, wv, bv,
        p['W_mg1'], row2(p['b_mg1']), p['W_mg2'].reshape(1, HD),
        p['b_mg2'].reshape(1, 1),
        wfc, p['b_fc'].reshape(1, 1),
        p['W_weight'], row2(p['b_weight']),
        row2(p['lnw_weight']), row2(p['lnb_weight']))


def kernel(corr_index, speed_index, angle_index, nei_index, hidden_state,
           cn, params):
    corr_t = corr_index.reshape(M, 2)
    speed_t = speed_index.reshape(M, 1)
    angle_t = angle_index.reshape(M, 1)
    maskf = (nei_index.reshape(M) > 0).astype(jnp.float32)
    hout, c = _run(corr_t, speed_t, angle_t, maskf.reshape(M, 1),
                   maskf.reshape(1, M), hidden_state, cn, params)
    any_mask = jnp.any(nei_index > 0)
    return (jnp.where(any_mask, hout, hidden_state),
            jnp.where(any_mask, c, cn))


# speedup vs baseline: 13.2494x; 1.0304x over previous
import jax
import jax.numpy as jnp
from jax.experimental import pallas as pl

N = 32
D = 64
HEADS = 4
OUT = 3 * D
HD = OUT // HEADS
M = N * N
_EPS = 1e-5
_NEG = -1e30


def _ln(x, w, b):
    u = jnp.mean(x, axis=-1, keepdims=True)
    xc = x - u
    s = jnp.mean(xc * xc, axis=-1, keepdims=True)
    return w * (xc * jax.lax.rsqrt(s + _EPS)) + b


def _fused_kernel(corr_ref, speed_ref, angle_ref, maskc_ref, maskr_ref,
                  hs_ref, cn_ref,
                  wr_ref, br_ref, lnwr_ref, lnbr_ref,
                  wsa_ref, bsa_ref, lnwsa_ref, lnbsa_ref,
                  wng_ref, bng_ref, lnwng_ref, lnbng_ref,
                  wq_ref, bq_ref, wk_ref, bk_ref, wv_ref, bv_ref,
                  wmg1_ref, bmg1_ref, wmg2_ref, bmg2_ref,
                  wfc_ref, bfc_ref,
                  ww_ref, bw_ref, lnww_ref, lnbw_ref,
                  hout_ref, cout_ref):
    corr = corr_ref[...]
    speed = speed_ref[...]
    angle = angle_ref[...]
    mask_c = maskc_ref[...]
    mask_r = maskr_ref[...]
    hs = hs_ref[...]

    m_col = jax.lax.broadcasted_iota(jnp.int32, (M, N), 0)
    j_col = jax.lax.broadcasted_iota(jnp.int32, (M, N), 1)
    tile_m = (jnp.remainder(m_col, N) == j_col).astype(jnp.float32)
    sel = ((m_col // N) == j_col).astype(jnp.float32)
    i_row = jax.lax.broadcasted_iota(jnp.int32, (N, M), 0)
    m_row = jax.lax.broadcasted_iota(jnp.int32, (N, M), 1)
    selt = (i_row == (m_row // N)).astype(jnp.float32)

    inp = jnp.dot(tile_m, hs, preferred_element_type=jnp.float32)
    hi = jnp.dot(sel, hs, preferred_element_type=jnp.float32)

    r_t = jnp.maximum(
        _ln(corr[:, 0:1] * wr_ref[0:1, :] + corr[:, 1:2] * wr_ref[1:2, :]
            + br_ref[...], lnwr_ref[...], lnbr_ref[...]), 0.0)
    s_t = jnp.maximum(
        _ln(speed * wsa_ref[...] + bsa_ref[...],
            lnwsa_ref[...], lnbsa_ref[...]), 0.0)
    a_t = jnp.maximum(
        _ln(angle * wsa_ref[...] + bsa_ref[...],
            lnwsa_ref[...], lnbsa_ref[...]), 0.0)

    parts = (r_t, s_t, a_t, hi, inp)

    acc = jnp.dot(parts[0], wng_ref[0], preferred_element_type=jnp.float32)
    for p in range(1, 5):
        acc = acc + jnp.dot(parts[p], wng_ref[p],
                            preferred_element_type=jnp.float32)
    ngate = jax.nn.sigmoid(_ln(acc + bng_ref[...],
                               lnwng_ref[...], lnbng_ref[...]))

    mg_h = jnp.maximum(
        speed * wmg1_ref[0:1, :] + angle * wmg1_ref[1:2, :] + bmg1_ref[...],
        0.0)
    mg = jax.nn.sigmoid(
        jnp.sum(mg_h * wmg2_ref[...], axis=1, keepdims=True)
        + bmg2_ref[...])

    scale = 1.0 / (HD ** 0.5)
    gate = mg * mask_c
    tt = jnp.zeros((M, 1), jnp.float32) + bfc_ref[...]
    for h in range(HEADS):
        qh = jnp.dot(parts[0], wq_ref[0, h], preferred_element_type=jnp.float32)
        kh = jnp.dot(parts[0], wk_ref[0, h], preferred_element_type=jnp.float32)
        vh = jnp.dot(parts[0], wv_ref[0, h], preferred_element_type=jnp.float32)
        for p in range(1, 5):
            qh = qh + jnp.dot(parts[p], wq_ref[p, h],
                              preferred_element_type=jnp.float32)
            kh = kh + jnp.dot(parts[p], wk_ref[p, h],
                              preferred_element_type=jnp.float32)
            vh = vh + jnp.dot(parts[p], wv_ref[p, h],
                              preferred_element_type=jnp.float32)
        qh = (qh + bq_ref[h]) * scale
        kh = kh + bk_ref[h]
        vh = vh + bv_ref[h]
        s = jax.lax.dot_general(qh, kh, (((1,), (1,)), ((), ())),
                                preferred_element_type=jnp.float32)
        mx = jnp.max(s, axis=1, keepdims=True)
        e = jnp.exp(s - mx)
        vg = jnp.concatenate((vh * gate, mask_c), axis=1)
        oh = jnp.dot(e, vg, preferred_element_type=jnp.float32)
        den = jnp.maximum(oh[:, HD:HD + 1], 1e-30)
        tt = tt + jnp.sum(oh[:, :HD] * wfc_ref[h], axis=1,
                          keepdims=True) / den

    pos0 = mask_c * tt
    pos = jnp.where(pos0 == 0.0, -10000.0, pos0)
    num = jnp.exp(pos)
    den_seg = jnp.dot(selt, num, preferred_element_type=jnp.float32)
    den_flat = jnp.dot(sel, den_seg, preferred_element_type=jnp.float32)
    pos_t = num / jnp.maximum(den_flat, 1e-30)

    hv = inp * ngate * pos_t
    hfull = mask_c * hv
    hsum = jnp.dot(selt, hfull, preferred_element_type=jnp.float32)
    hsum = jnp.maximum(
        _ln(jnp.dot(hsum, ww_ref[...], preferred_element_type=jnp.float32)
            + bw_ref[...], lnww_ref[...], lnbw_ref[...]), 0.0)
    c = hsum + cn_ref[...]
    cout_ref[...] = c
    hout_ref[...] = hs + jnp.tanh(c)


def _run(corr_t, speed_t, angle_t, mask_c, mask_r, hidden_state, cn, p,
         interpret=False):
    row2 = lambda v: v.reshape(1, -1)
    wq = p['W_q'].reshape(5, D, HEADS, HD).transpose(0, 2, 1, 3)
    wk = p['W_k'].reshape(5, D, HEADS, HD).transpose(0, 2, 1, 3)
    wv = p['W_v'].reshape(5, D, HEADS, HD).transpose(0, 2, 1, 3)
    bq = p['b_q'].reshape(HEADS, 1, HD)
    bk = p['b_k'].reshape(HEADS, 1, HD)
    bv = p['b_v'].reshape(HEADS, 1, HD)
    wfc = p['W_fc'][:, 0].reshape(HEADS, 1, HD)
    out_sds = (jax.ShapeDtypeStruct((N, D), jnp.float32),
               jax.ShapeDtypeStruct((N, D), jnp.float32))
    return pl.pallas_call(_fused_kernel, out_shape=out_sds,
                          interpret=interpret)(
        corr_t, speed_t, angle_t, mask_c, mask_r, hidden_state, cn,
        p['W_r'], row2(p['b_r']), row2(p['lnw_r']), row2(p['lnb_r']),
        p['W_sa'], row2(p['b_sa']), row2(p['lnw_sa']), row2(p['lnb_sa']),
        p['W_ngate'].reshape(5, D, D), row2(p['b_ngate']),
        row2(p['lnw_ngate']), row2(p['lnb_ngate']),
        wq, bq, wk, bk, wv, bv,
        p['W_mg1'], row2(p['b_mg1']), p['W_mg2'].reshape(1, HD),
        p['b_mg2'].reshape(1, 1),
        wfc, p['b_fc'].reshape(1, 1),
        p['W_weight'], row2(p['b_weight']),
        row2(p['lnw_weight']), row2(p['lnb_weight']))


def kernel(corr_index, speed_index, angle_index, nei_index, hidden_state,
           cn, params):
    corr_t = corr_index.reshape(M, 2)
    speed_t = speed_index.reshape(M, 1)
    angle_t = angle_index.reshape(M, 1)
    maskf = (nei_index.reshape(M) > 0).astype(jnp.float32)
    hout, c = _run(corr_t, speed_t, angle_t, maskf.reshape(M, 1),
                   maskf.reshape(1, M), hidden_state, cn, params)
    any_mask = jnp.any(nei_index > 0)
    return (jnp.where(any_mask, hout, hidden_state),
            jnp.where(any_mask, c, cn))


# speedup vs baseline: 17.3572x; 1.3100x over previous
import jax
import jax.numpy as jnp
from jax.experimental import pallas as pl

N = 32
D = 64
HEADS = 4
OUT = 3 * D
HD = OUT // HEADS
M = N * N
_EPS = 1e-5

_B_R, _LNW_R, _LNB_R = 0, 1, 2
_B_SA, _LNW_SA, _LNB_SA = 3, 4, 5
_B_NG, _LNW_NG, _LNB_NG = 6, 7, 8
_B_W, _LNW_W, _LNB_W = 9, 10, 11


def _ln(x, w, b):
    u = jnp.mean(x, axis=-1, keepdims=True)
    xc = x - u
    s = jnp.mean(xc * xc, axis=-1, keepdims=True)
    return w * (xc * jax.lax.rsqrt(s + _EPS)) + b


def _fused_kernel(feat_ref, hs_ref, cn_ref,
                  wr_ref, wsa_ref, vec_ref,
                  wng_ref, wq_ref, wk_ref, wv_ref, bqkv_ref,
                  wmg1_ref, wmg2_ref, wfc_ref, scal_ref,
                  ww_ref,
                  hout_ref, cout_ref):
    corr0 = feat_ref[:, 0:1]
    corr1 = feat_ref[:, 1:2]
    speed = feat_ref[:, 2:3]
    angle = feat_ref[:, 3:4]
    mask_c = feat_ref[:, 4:5]
    hs = hs_ref[...]

    def vec(i):
        return vec_ref[i:i + 1, :]

    m_col = jax.lax.broadcasted_iota(jnp.int32, (M, N), 0)
    j_col = jax.lax.broadcasted_iota(jnp.int32, (M, N), 1)
    tile_m = (jnp.remainder(m_col, N) == j_col).astype(jnp.float32)
    sel = ((m_col // N) == j_col).astype(jnp.float32)
    i_row = jax.lax.broadcasted_iota(jnp.int32, (N, M), 0)
    m_row = jax.lax.broadcasted_iota(jnp.int32, (N, M), 1)
    selt = (i_row == (m_row // N)).astype(jnp.float32)

    inp = jnp.dot(tile_m, hs, preferred_element_type=jnp.float32)
    hi = jnp.dot(sel, hs, preferred_element_type=jnp.float32)

    r_t = jnp.maximum(
        _ln(corr0 * wr_ref[0:1, :] + corr1 * wr_ref[1:2, :] + vec(_B_R),
            vec(_LNW_R), vec(_LNB_R)), 0.0)
    s_t = jnp.maximum(
        _ln(speed * wsa_ref[...] + vec(_B_SA), vec(_LNW_SA), vec(_LNB_SA)),
        0.0)
    a_t = jnp.maximum(
        _ln(angle * wsa_ref[...] + vec(_B_SA), vec(_LNW_SA), vec(_LNB_SA)),
        0.0)

    parts = (r_t, s_t, a_t, hi, inp)

    def proj(w_ref, width):
        acc = jnp.dot(parts[0], w_ref[0:D, :],
                      preferred_element_type=jnp.float32)
        for p in range(1, 5):
            acc = acc + jnp.dot(parts[p], w_ref[p * D:(p + 1) * D, :],
                                preferred_element_type=jnp.float32)
        return acc

    ngate = jax.nn.sigmoid(_ln(proj(wng_ref, D) + vec(_B_NG),
                               vec(_LNW_NG), vec(_LNB_NG)))

    mg_h = jnp.maximum(
        speed * wmg1_ref[0:1, :] + angle * wmg1_ref[1:2, :] + wmg1_ref[2:3, :],
        0.0)
    mg = jax.nn.sigmoid(
        jnp.dot(mg_h, wmg2_ref[...], preferred_element_type=jnp.float32)
        + scal_ref[0:1, 0:1])

    qf = (proj(wq_ref, OUT) + bqkv_ref[0:1, :]) * (1.0 / (HD ** 0.5))
    kf = proj(wk_ref, OUT) + bqkv_ref[1:2, :]
    vf = proj(wv_ref, OUT) + bqkv_ref[2:3, :]

    gate = mg * mask_c
    tt = jnp.zeros((M, 1), jnp.float32) + scal_ref[0:1, 1:2]
    for h in range(HEADS):
        qh = qf[:, h * HD:(h + 1) * HD]
        kh = kf[:, h * HD:(h + 1) * HD]
        vh = vf[:, h * HD:(h + 1) * HD]
        e = jnp.exp(jax.lax.dot_general(qh, kh, (((1,), (1,)), ((), ())),
                                        preferred_element_type=jnp.float32))
        vg = jnp.concatenate((vh * gate, mask_c), axis=1)
        oh = jnp.dot(e, vg, preferred_element_type=jnp.float32)
        den = jnp.maximum(oh[:, HD:HD + 1], 1e-30)
        tt = tt + jnp.dot(oh[:, 0:HD], wfc_ref[h * HD:(h + 1) * HD, :],
                          preferred_element_type=jnp.float32) / den

    pos0 = mask_c * tt
    pos = jnp.where(pos0 == 0.0, -10000.0, pos0)
    num = jnp.exp(pos)
    den_seg = jnp.dot(selt, num, preferred_element_type=jnp.float32)
    den_flat = jnp.dot(sel, den_seg, preferred_element_type=jnp.float32)
    pos_t = num / jnp.maximum(den_flat, 1e-30)

    hv = inp * ngate * pos_t
    hfull = mask_c * hv
    hsum = jnp.dot(selt, hfull, preferred_element_type=jnp.float32)
    hsum = jnp.maximum(
        _ln(jnp.dot(hsum, ww_ref[...], preferred_element_type=jnp.float32)
            + vec(_B_W), vec(_LNW_W), vec(_LNB_W)), 0.0)
    c = hsum + cn_ref[...]

    flag = (jnp.max(mask_c) > 0).astype(jnp.float32)
    cout_ref[...] = flag * c + (1.0 - flag) * cn_ref[...]
    hout_ref[...] = hs + flag * jnp.tanh(c)


def _run(feat, hidden_state, cn, p, interpret=False):
    vec64 = jnp.stack((p['b_r'], p['lnw_r'], p['lnb_r'],
                       p['b_sa'], p['lnw_sa'], p['lnb_sa'],
                       p['b_ngate'], p['lnw_ngate'], p['lnb_ngate'],
                       p['b_weight'], p['lnw_weight'], p['lnb_weight']))
    bqkv = jnp.stack((p['b_q'], p['b_k'], p['b_v']))
    wmg1 = jnp.concatenate((p['W_mg1'], p['b_mg1'][None, :]))
    scal = jnp.stack((p['b_mg2'][0], p['b_fc'][0])).reshape(1, 2)
    out_sds = (jax.ShapeDtypeStruct((N, D), jnp.float32),
               jax.ShapeDtypeStruct((N, D), jnp.float32))
    return pl.pallas_call(_fused_kernel, out_shape=out_sds,
                          interpret=interpret)(
        feat, hidden_state, cn,
        p['W_r'], p['W_sa'], vec64,
        p['W_ngate'], p['W_q'], p['W_k'], p['W_v'], bqkv,
        wmg1, p['W_mg2'], p['W_fc'], scal,
        p['W_weight'])


def kernel(corr_index, speed_index, angle_index, nei_index, hidden_state,
           cn, params):
    feat = jnp.concatenate(
        (corr_index.reshape(M, 2), speed_index.reshape(M, 1),
         angle_index.reshape(M, 1),
         (nei_index.reshape(M, 1) > 0).astype(jnp.float32)), axis=1)
    return _run(feat, hidden_state, cn, params)


# speedup vs baseline: 18.8985x; 1.0888x over previous
import jax
import jax.numpy as jnp
from jax.experimental import pallas as pl

N = 32
D = 64
HEADS = 4
OUT = 3 * D
HD = OUT // HEADS
M = N * N
_EPS = 1e-5

_B_R, _LNW_R, _LNB_R = 0, 1, 2
_B_SA, _LNW_SA, _LNB_SA = 3, 4, 5
_B_NG, _LNW_NG, _LNB_NG = 6, 7, 8
_B_W, _LNW_W, _LNB_W = 9, 10, 11


def _ln(x, w, b):
    m = x.shape[0]
    ones_col = jnp.ones((x.shape[1], 1), jnp.float32)
    s1 = jnp.dot(jnp.concatenate((x, x * x), axis=0), ones_col,
                 preferred_element_type=jnp.float32) * (1.0 / x.shape[1])
    u = s1[0:m]
    var = s1[m:2 * m] - u * u
    return w * ((x - u) * jax.lax.rsqrt(var + _EPS)) + b


def _fused_kernel(feat_ref, hs_ref, cn_ref,
                  wr_ref, wsa_ref, vec_ref,
                  wng_ref, wq_ref, wk_ref, wv_ref, bqkv_ref,
                  wmg1_ref, wmg2_ref, wfc_ref, scal_ref,
                  ww_ref,
                  hout_ref, cout_ref):
    corr0 = feat_ref[:, 0:1]
    corr1 = feat_ref[:, 1:2]
    speed = feat_ref[:, 2:3]
    angle = feat_ref[:, 3:4]
    mask_c = feat_ref[:, 4:5]
    hs = hs_ref[...]

    def vec(i):
        return vec_ref[i:i + 1, :]

    m_col = jax.lax.broadcasted_iota(jnp.int32, (M, N), 0)
    j_col = jax.lax.broadcasted_iota(jnp.int32, (M, N), 1)
    tile_m = (jnp.remainder(m_col, N) == j_col).astype(jnp.float32)
    sel = ((m_col // N) == j_col).astype(jnp.float32)
    i_row = jax.lax.broadcasted_iota(jnp.int32, (N, M), 0)
    m_row = jax.lax.broadcasted_iota(jnp.int32, (N, M), 1)
    selt = (i_row == (m_row // N)).astype(jnp.float32)

    inp = jnp.dot(tile_m, hs, preferred_element_type=jnp.float32)
    hi = jnp.dot(sel, hs, preferred_element_type=jnp.float32)

    r_t = jnp.maximum(
        _ln(corr0 * wr_ref[0:1, :] + corr1 * wr_ref[1:2, :] + vec(_B_R),
            vec(_LNW_R), vec(_LNB_R)), 0.0)
    s_t = jnp.maximum(
        _ln(speed * wsa_ref[...] + vec(_B_SA), vec(_LNW_SA), vec(_LNB_SA)),
        0.0)
    a_t = jnp.maximum(
        _ln(angle * wsa_ref[...] + vec(_B_SA), vec(_LNW_SA), vec(_LNB_SA)),
        0.0)

    parts = (r_t, s_t, a_t, hi, inp)

    def proj(w_ref, width):
        acc = jnp.dot(parts[0], w_ref[0:D, :],
                      preferred_element_type=jnp.float32)
        for p in range(1, 5):
            acc = acc + jnp.dot(parts[p], w_ref[p * D:(p + 1) * D, :],
                                preferred_element_type=jnp.float32)
        return acc

    ngate = jax.nn.sigmoid(_ln(proj(wng_ref, D) + vec(_B_NG),
                               vec(_LNW_NG), vec(_LNB_NG)))

    mg_h = jnp.maximum(
        speed * wmg1_ref[0:1, :] + angle * wmg1_ref[1:2, :] + wmg1_ref[2:3, :],
        0.0)
    mg = jax.nn.sigmoid(
        jnp.dot(mg_h, wmg2_ref[...], preferred_element_type=jnp.float32)
        + scal_ref[0:1, 0:1])

    qf = (proj(wq_ref, OUT) + bqkv_ref[0:1, :]) * (1.0 / (HD ** 0.5))
    kf = proj(wk_ref, OUT) + bqkv_ref[1:2, :]

    wvf_p = []
    for p in range(5):
        cols = [jnp.dot(wv_ref[p * D:(p + 1) * D, h * HD:(h + 1) * HD],
                        wfc_ref[h * HD:(h + 1) * HD, :],
                        preferred_element_type=jnp.float32)
                for h in range(HEADS)]
        wvf_p.append(jnp.concatenate(cols, axis=1))
    bvf = jnp.concatenate(
        [jnp.dot(bqkv_ref[2:3, h * HD:(h + 1) * HD],
                 wfc_ref[h * HD:(h + 1) * HD, :],
                 preferred_element_type=jnp.float32)
         for h in range(HEADS)], axis=1)
    uval = jnp.dot(parts[0], wvf_p[0], preferred_element_type=jnp.float32)
    for p in range(1, 5):
        uval = uval + jnp.dot(parts[p], wvf_p[p],
                              preferred_element_type=jnp.float32)
    gate = mg * mask_c
    u5 = jnp.concatenate(((uval + bvf) * gate, mask_c), axis=1)

    tt = jnp.zeros((M, 1), jnp.float32) + scal_ref[0:1, 1:2]
    for h in range(HEADS):
        qh = qf[:, h * HD:(h + 1) * HD]
        kh = kf[:, h * HD:(h + 1) * HD]
        e = jnp.exp(jax.lax.dot_general(qh, kh, (((1,), (1,)), ((), ())),
                                        preferred_element_type=jnp.float32))
        oh = jnp.dot(e, u5, preferred_element_type=jnp.float32)
        tt = tt + oh[:, h:h + 1] / jnp.maximum(oh[:, HEADS:HEADS + 1],
                                               1e-30)

    pos0 = mask_c * tt
    pos = jnp.where(pos0 == 0.0, -10000.0, pos0)
    num = jnp.exp(pos)
    den_seg = jnp.dot(selt, num, preferred_element_type=jnp.float32)
    den_flat = jnp.dot(sel, den_seg, preferred_element_type=jnp.float32)
    pos_t = num / jnp.maximum(den_flat, 1e-30)

    hv = inp * ngate * pos_t
    hfull = mask_c * hv
    hsum = jnp.dot(selt, hfull, preferred_element_type=jnp.float32)
    hsum = jnp.maximum(
        _ln(jnp.dot(hsum, ww_ref[...], preferred_element_type=jnp.float32)
            + vec(_B_W), vec(_LNW_W), vec(_LNB_W)), 0.0)
    c = hsum + cn_ref[...]

    flag = (jnp.max(mask_c) > 0).astype(jnp.float32)
    cout_ref[...] = flag * c + (1.0 - flag) * cn_ref[...]
    hout_ref[...] = hs + flag * jnp.tanh(c)


def _run(feat, hidden_state, cn, p, interpret=False):
    vec64 = jnp.stack((p['b_r'], p['lnw_r'], p['lnb_r'],
                       p['b_sa'], p['lnw_sa'], p['lnb_sa'],
                       p['b_ngate'], p['lnw_ngate'], p['lnb_ngate'],
                       p['b_weight'], p['lnw_weight'], p['lnb_weight']))
    bqkv = jnp.stack((p['b_q'], p['b_k'], p['b_v']))
    wmg1 = jnp.concatenate((p['W_mg1'], p['b_mg1'][None, :]))
    scal = jnp.stack((p['b_mg2'][0], p['b_fc'][0])).reshape(1, 2)
    out_sds = (jax.ShapeDtypeStruct((N, D), jnp.float32),
               jax.ShapeDtypeStruct((N, D), jnp.float32))
    return pl.pallas_call(_fused_kernel, out_shape=out_sds,
                          interpret=interpret)(
        feat, hidden_state, cn,
        p['W_r'], p['W_sa'], vec64,
        p['W_ngate'], p['W_q'], p['W_k'], p['W_v'], bqkv,
        wmg1, p['W_mg2'], p['W_fc'], scal,
        p['W_weight'])


def kernel(corr_index, speed_index, angle_index, nei_index, hidden_state,
           cn, params):
    feat = jnp.concatenate(
        (corr_index.reshape(M, 2), speed_index.reshape(M, 1),
         angle_index.reshape(M, 1),
         (nei_index.reshape(M, 1) > 0).astype(jnp.float32)), axis=1)
    return _run(feat, hidden_state, cn, params)


# speedup vs baseline: 20.6107x; 1.0906x over previous
import jax
import jax.numpy as jnp
from jax.experimental import pallas as pl

N = 32
D = 64
HEADS = 4
OUT = 3 * D
HD = OUT // HEADS
M = N * N
_EPS = 1e-5


def _ln(x, w, b):
    m = x.shape[0]
    ones_col = jnp.ones((x.shape[1], 1), jnp.float32)
    s1 = jnp.dot(jnp.concatenate((x, x * x), axis=0), ones_col,
                 preferred_element_type=jnp.float32) * (1.0 / x.shape[1])
    u = s1[0:m]
    var = s1[m:2 * m] - u * u
    return w * ((x - u) * jax.lax.rsqrt(var + _EPS)) + b


def _fused_kernel(corr_ref, speed_ref, angle_ref, nei_ref, hs_ref, cn_ref,
                  wr_ref, br_ref, lnwr_ref, lnbr_ref,
                  wsa_ref, bsa_ref, lnwsa_ref, lnbsa_ref,
                  wng_ref, bng_ref, lnwng_ref, lnbng_ref,
                  wq_ref, bq_ref, wk_ref, bk_ref, wv_ref, bv_ref,
                  wmg1_ref, bmg1_ref, wmg2_ref, bmg2_ref,
                  wfc_ref, bfc_ref, ww_ref, bw_ref, lnww_ref, lnbw_ref,
                  hout_ref, cout_ref):
    corr = corr_ref[...].reshape(M, 2)
    speed = speed_ref[...].reshape(M, 1)
    angle = angle_ref[...].reshape(M, 1)
    hs = hs_ref[...]

    def row(r):
        return r[...].reshape(1, -1)

    corr0 = corr[:, 0:1]
    corr1 = corr[:, 1:2]

    m_col = jax.lax.broadcasted_iota(jnp.int32, (M, N), 0)
    j_col = jax.lax.broadcasted_iota(jnp.int32, (M, N), 1)
    tile_m = (jnp.remainder(m_col, N) == j_col).astype(jnp.float32)
    sel = ((m_col // N) == j_col).astype(jnp.float32)
    i_row = jax.lax.broadcasted_iota(jnp.int32, (N, M), 0)
    m_row = jax.lax.broadcasted_iota(jnp.int32, (N, M), 1)
    selt = (i_row == (m_row // N)).astype(jnp.float32)

    mask32 = (nei_ref[...] > 0).astype(jnp.float32)
    ones_n = jnp.ones((N, 1), jnp.float32)
    mask_c = jnp.dot(
        jnp.dot(sel, mask32, preferred_element_type=jnp.float32) * tile_m,
        ones_n, preferred_element_type=jnp.float32)

    inp = jnp.dot(tile_m, hs, preferred_element_type=jnp.float32)
    hi = jnp.dot(sel, hs, preferred_element_type=jnp.float32)

    r_t = jnp.maximum(
        _ln(corr0 * wr_ref[0:1, :] + corr1 * wr_ref[1:2, :] + row(br_ref),
            row(lnwr_ref), row(lnbr_ref)), 0.0)
    s_t = jnp.maximum(
        _ln(speed * wsa_ref[...] + row(bsa_ref),
            row(lnwsa_ref), row(lnbsa_ref)), 0.0)
    a_t = jnp.maximum(
        _ln(angle * wsa_ref[...] + row(bsa_ref),
            row(lnwsa_ref), row(lnbsa_ref)), 0.0)

    parts = (r_t, s_t, a_t, hi, inp)

    def proj(w_ref):
        acc = jnp.dot(parts[0], w_ref[0:D, :],
                      preferred_element_type=jnp.float32)
        for p in range(1, 5):
            acc = acc + jnp.dot(parts[p], w_ref[p * D:(p + 1) * D, :],
                                preferred_element_type=jnp.float32)
        return acc

    ngate = jax.nn.sigmoid(_ln(proj(wng_ref) + row(bng_ref),
                               row(lnwng_ref), row(lnbng_ref)))

    mg_h = jnp.maximum(
        speed * wmg1_ref[0:1, :] + angle * wmg1_ref[1:2, :] + row(bmg1_ref),
        0.0)
    mg = jax.nn.sigmoid(
        jnp.dot(mg_h, wmg2_ref[...], preferred_element_type=jnp.float32)
        + bmg2_ref[...].reshape(1, 1))

    qf = (proj(wq_ref) + row(bq_ref)) * (1.0 / (HD ** 0.5))
    kf = proj(wk_ref) + row(bk_ref)

    bv_row = row(bv_ref)
    wvf_p = []
    for p in range(5):
        cols = [jnp.dot(wv_ref[p * D:(p + 1) * D, h * HD:(h + 1) * HD],
                        wfc_ref[h * HD:(h + 1) * HD, :],
                        preferred_element_type=jnp.float32)
                for h in range(HEADS)]
        wvf_p.append(jnp.concatenate(cols, axis=1))
    bvf = jnp.concatenate(
        [jnp.dot(bv_row[:, h * HD:(h + 1) * HD],
                 wfc_ref[h * HD:(h + 1) * HD, :],
                 preferred_element_type=jnp.float32)
         for h in range(HEADS)], axis=1)
    uval = jnp.dot(parts[0], wvf_p[0], preferred_element_type=jnp.float32)
    for p in range(1, 5):
        uval = uval + jnp.dot(parts[p], wvf_p[p],
                              preferred_element_type=jnp.float32)
    gate = mg * mask_c
    u5 = jnp.concatenate(((uval + bvf) * gate, mask_c), axis=1)

    tt = jnp.zeros((M, 1), jnp.float32) + bfc_ref[...].reshape(1, 1)
    for h in range(HEADS):
        qh = qf[:, h * HD:(h + 1) * HD]
        kh = kf[:, h * HD:(h + 1) * HD]
        e = jnp.exp(jax.lax.dot_general(qh, kh, (((1,), (1,)), ((), ())),
                                        preferred_element_type=jnp.float32))
        oh = jnp.dot(e, u5, preferred_element_type=jnp.float32)
        tt = tt + oh[:, h:h + 1] / jnp.maximum(oh[:, HEADS:HEADS + 1],
                                               1e-30)

    pos0 = mask_c * tt
    pos = jnp.where(pos0 == 0.0, -10000.0, pos0)
    num = jnp.exp(pos)
    den_seg = jnp.dot(selt, num, preferred_element_type=jnp.float32)
    den_flat = jnp.dot(sel, den_seg, preferred_element_type=jnp.float32)
    pos_t = num / jnp.maximum(den_flat, 1e-30)

    hv = inp * ngate * pos_t
    hfull = mask_c * hv
    hsum = jnp.dot(selt, hfull, preferred_element_type=jnp.float32)
    hsum = jnp.maximum(
        _ln(jnp.dot(hsum, ww_ref[...], preferred_element_type=jnp.float32)
            + row(bw_ref), row(lnww_ref), row(lnbw_ref)), 0.0)
    c = hsum + cn_ref[...]

    flag = (jnp.max(mask_c) > 0).astype(jnp.float32)
    cout_ref[...] = flag * c + (1.0 - flag) * cn_ref[...]
    hout_ref[...] = hs + flag * jnp.tanh(c)


def _run(corr_index, speed_index, angle_index, nei_index, hidden_state, cn,
         p, interpret=False):
    out_sds = (jax.ShapeDtypeStruct((N, D), jnp.float32),
               jax.ShapeDtypeStruct((N, D), jnp.float32))
    return pl.pallas_call(_fused_kernel, out_shape=out_sds,
                          interpret=interpret)(
        corr_index, speed_index, angle_index, nei_index, hidden_state, cn,
        p['W_r'], p['b_r'], p['lnw_r'], p['lnb_r'],
        p['W_sa'], p['b_sa'], p['lnw_sa'], p['lnb_sa'],
        p['W_ngate'], p['b_ngate'], p['lnw_ngate'], p['lnb_ngate'],
        p['W_q'], p['b_q'], p['W_k'], p['b_k'], p['W_v'], p['b_v'],
        p['W_mg1'], p['b_mg1'], p['W_mg2'], p['b_mg2'],
        p['W_fc'], p['b_fc'], p['W_weight'], p['b_weight'],
        p['lnw_weight'], p['lnb_weight'])


def kernel(corr_index, speed_index, angle_index, nei_index, hidden_state,
           cn, params):
    return _run(corr_index, speed_index, angle_index, nei_index,
                hidden_state, cn, params)


# speedup vs baseline: 24.3120x; 1.1796x over previous
import jax
import jax.numpy as jnp
from jax.experimental import pallas as pl

N = 32
D = 64
HEADS = 4
OUT = 3 * D
HD = OUT // HEADS
M = N * N
_EPS = 1e-5


def _ln(x, w, b):
    m = x.shape[0]
    ones_col = jnp.ones((x.shape[1], 1), jnp.float32)
    s1 = jnp.dot(jnp.concatenate((x, x * x), axis=0), ones_col,
                 preferred_element_type=jnp.float32) * (1.0 / x.shape[1])
    u = s1[0:m]
    var = s1[m:2 * m] - u * u
    return w * ((x - u) * jax.lax.rsqrt(var + _EPS)) + b


def _fused_kernel(pf_ref, hs_ref, cn_ref,
                  wr_ref, br_ref, lnwr_ref, lnbr_ref,
                  wsa_ref, bsa_ref, lnwsa_ref, lnbsa_ref,
                  wng_ref, bng_ref, lnwng_ref, lnbng_ref,
                  wq_ref, bq_ref, wk_ref, bk_ref, wv_ref, bv_ref,
                  wmg1_ref, bmg1_ref, wmg2_ref, bmg2_ref,
                  wfc_ref, bfc_ref, ww_ref, bw_ref, lnww_ref, lnbw_ref,
                  hout_ref, cout_ref):
    hs = hs_ref[...]

    def row(r):
        return r[...].reshape(1, -1)

    m_col = jax.lax.broadcasted_iota(jnp.int32, (M, N), 0)
    j_col = jax.lax.broadcasted_iota(jnp.int32, (M, N), 1)
    tile_m = (jnp.remainder(m_col, N) == j_col).astype(jnp.float32)
    sel = ((m_col // N) == j_col).astype(jnp.float32)
    i_row = jax.lax.broadcasted_iota(jnp.int32, (N, M), 0)
    m_row = jax.lax.broadcasted_iota(jnp.int32, (N, M), 1)
    selt = (i_row == (m_row // N)).astype(jnp.float32)

    xcat = jnp.concatenate([pf_ref[i] for i in range(5)], axis=1)
    big = jnp.dot(sel, xcat, preferred_element_type=jnp.float32)
    l_col = jax.lax.broadcasted_iota(jnp.int32, (M, 5 * N), 1)
    tile5 = (jnp.remainder(m_col[:, 0:1], N)
             == jnp.remainder(l_col, N)).astype(jnp.float32)
    b_row = jax.lax.broadcasted_iota(jnp.int32, (5 * N, 5), 0)
    b_col = jax.lax.broadcasted_iota(jnp.int32, (5 * N, 5), 1)
    blk5 = ((b_row // N) == b_col).astype(jnp.float32)
    flats = jnp.dot(big * tile5, blk5,
                    preferred_element_type=jnp.float32)
    corr0 = flats[:, 0:1]
    corr1 = flats[:, 1:2]
    speed = flats[:, 2:3]
    angle = flats[:, 3:4]
    mask_c = flats[:, 4:5]

    inp = jnp.dot(tile_m, hs, preferred_element_type=jnp.float32)
    hi = jnp.dot(sel, hs, preferred_element_type=jnp.float32)

    r_t = jnp.maximum(
        _ln(corr0 * wr_ref[0:1, :] + corr1 * wr_ref[1:2, :] + row(br_ref),
            row(lnwr_ref), row(lnbr_ref)), 0.0)
    s_t = jnp.maximum(
        _ln(speed * wsa_ref[...] + row(bsa_ref),
            row(lnwsa_ref), row(lnbsa_ref)), 0.0)
    a_t = jnp.maximum(
        _ln(angle * wsa_ref[...] + row(bsa_ref),
            row(lnwsa_ref), row(lnbsa_ref)), 0.0)

    parts = (r_t, s_t, a_t, hi, inp)

    def proj(w_ref):
        acc = jnp.dot(parts[0], w_ref[0:D, :],
                      preferred_element_type=jnp.float32)
        for p in range(1, 5):
            acc = acc + jnp.dot(parts[p], w_ref[p * D:(p + 1) * D, :],
                                preferred_element_type=jnp.float32)
        return acc

    ngate = jax.nn.sigmoid(_ln(proj(wng_ref) + row(bng_ref),
                               row(lnwng_ref), row(lnbng_ref)))

    mg_h = jnp.maximum(
        speed * wmg1_ref[0:1, :] + angle * wmg1_ref[1:2, :] + row(bmg1_ref),
        0.0)
    mg = jax.nn.sigmoid(
        jnp.dot(mg_h, wmg2_ref[...], preferred_element_type=jnp.float32)
        + bmg2_ref[...].reshape(1, 1))

    qf = (proj(wq_ref) + row(bq_ref)) * (1.0 / (HD ** 0.5))
    kf = proj(wk_ref) + row(bk_ref)

    bv_row = row(bv_ref)
    wvf_p = []
    for p in range(5):
        cols = [jnp.dot(wv_ref[p * D:(p + 1) * D, h * HD:(h + 1) * HD],
                        wfc_ref[h * HD:(h + 1) * HD, :],
                        preferred_element_type=jnp.float32)
                for h in range(HEADS)]
        wvf_p.append(jnp.concatenate(cols, axis=1))
    bvf = jnp.concatenate(
        [jnp.dot(bv_row[:, h * HD:(h + 1) * HD],
                 wfc_ref[h * HD:(h + 1) * HD, :],
                 preferred_element_type=jnp.float32)
         for h in range(HEADS)], axis=1)
    uval = jnp.dot(parts[0], wvf_p[0], preferred_element_type=jnp.float32)
    for p in range(1, 5):
        uval = uval + jnp.dot(parts[p], wvf_p[p],
                              preferred_element_type=jnp.float32)
    gate = mg * mask_c
    u5 = jnp.concatenate(((uval + bvf) * gate, mask_c), axis=1)

    tt = jnp.zeros((M, 1), jnp.float32) + bfc_ref[...].reshape(1, 1)
    for h in range(HEADS):
        qh = qf[:, h * HD:(h + 1) * HD]
        kh = kf[:, h * HD:(h + 1) * HD]
        e = jnp.exp(jax.lax.dot_general(qh, kh, (((1,), (1,)), ((), ())),
                                        preferred_element_type=jnp.float32))
        oh = jnp.dot(e, u5, preferred_element_type=jnp.float32)
        tt = tt + oh[:, h:h + 1] / jnp.maximum(oh[:, HEADS:HEADS + 1],
                                               1e-30)

    pos0 = mask_c * tt
    pos = jnp.where(pos0 == 0.0, -10000.0, pos0)
    num = jnp.exp(pos)
    den_seg = jnp.dot(selt, num, preferred_element_type=jnp.float32)
    den_flat = jnp.dot(sel, den_seg, preferred_element_type=jnp.float32)
    pos_t = num / jnp.maximum(den_flat, 1e-30)

    hv = inp * ngate * pos_t
    hfull = mask_c * hv
    hsum = jnp.dot(selt, hfull, preferred_element_type=jnp.float32)
    hsum = jnp.maximum(
        _ln(jnp.dot(hsum, ww_ref[...], preferred_element_type=jnp.float32)
            + row(bw_ref), row(lnww_ref), row(lnbw_ref)), 0.0)
    c = hsum + cn_ref[...]

    flag = (jnp.max(mask_c) > 0).astype(jnp.float32)
    cout_ref[...] = flag * c + (1.0 - flag) * cn_ref[...]
    hout_ref[...] = hs + flag * jnp.tanh(c)


def _run(corr_index, speed_index, angle_index, nei_index, hidden_state, cn,
         p, interpret=False):
    pf = jnp.stack((corr_index[:, :, 0], corr_index[:, :, 1],
                    speed_index[:, :, 0], angle_index[:, :, 0],
                    (nei_index > 0).astype(jnp.float32)))
    out_sds = (jax.ShapeDtypeStruct((N, D), jnp.float32),
               jax.ShapeDtypeStruct((N, D), jnp.float32))
    return pl.pallas_call(_fused_kernel, out_shape=out_sds,
                          interpret=interpret)(
        pf, hidden_state, cn,
        p['W_r'], p['b_r'], p['lnw_r'], p['lnb_r'],
        p['W_sa'], p['b_sa'], p['lnw_sa'], p['lnb_sa'],
        p['W_ngate'], p['b_ngate'], p['lnw_ngate'], p['lnb_ngate'],
        p['W_q'], p['b_q'], p['W_k'], p['b_k'], p['W_v'], p['b_v'],
        p['W_mg1'], p['b_mg1'], p['W_mg2'], p['b_mg2'],
        p['W_fc'], p['b_fc'], p['W_weight'], p['b_weight'],
        p['lnw_weight'], p['lnb_weight'])


def kernel(corr_index, speed_index, angle_index, nei_index, hidden_state,
           cn, params):
    return _run(corr_index, speed_index, angle_index, nei_index,
                hidden_state, cn, params)
